# baseline (speedup 1.0000x reference)
import functools

import jax
import jax.numpy as jnp
from jax import lax
from jax.experimental import pallas as pl
from jax.experimental.pallas import tpu as pltpu
from jax.experimental.pallas import tpu_sc as plsc

_B = 1024
_N = 200
_E = 64
_L = 16
_NTILES = 32
_EPT = _B // _NTILES
_G = 8
_NG = _EPT // _G


def _splat_i(x):
    return jnp.full((_L,), x, dtype=jnp.int32)


def _tree_kernel(subtree_hbm, embind_hbm, slens_hbm, emb_hbm, wlr_hbm,
                 btree_hbm, wcls_hbm, bcls_hbm, out_hbm,
                 reps_v, subtree_v, embind_v, slens_v, wlr_v, btree_v,
                 wcls_v, bcls_v, out_stage, sem):
    nc = 2
    wid = lax.axis_index("s") * nc + lax.axis_index("c")

    pltpu.sync_copy(wlr_hbm, wlr_v)
    pltpu.sync_copy(btree_hbm, btree_v)
    pltpu.sync_copy(wcls_hbm, wcls_v)
    pltpu.sync_copy(bcls_hbm, bcls_v)

    iota = lax.iota(jnp.int32, _L)

    def group_body(g, carry):
        e0 = wid * _EPT + g * _G
        grp = wid * _NG + g

        pltpu.sync_copy(subtree_hbm.at[pl.ds(e0 * (_N * 3), _G * _N * 3)],
                        subtree_v.at[pl.ds(0, _G * _N * 3)])
        pltpu.sync_copy(embind_hbm.at[grp], embind_v)
        pltpu.sync_copy(slens_hbm.at[grp], slens_v.at[pl.ds(0, _L)])

        for j in range(16):
            pltpu.async_copy(emb_hbm.at[embind_v.at[j]],
                             reps_v.at[pl.ds(j * 100, 100)], sem).wait()

        def pair_body(p, carry2):
            sl = slens_v[pl.ds(2 * p, _L)]
            s0 = sl[0]
            s1 = sl[1]
            row0a = (2 * p) * _N
            row0b = (2 * p + 1) * _N
            sba = (2 * p) * (_N * 3)
            sbb = (2 * p + 1) * (_N * 3)
            s0v = _splat_i(s0)
            s1v = _splat_i(s1)

            def step_body(i, carry3):
                iba = _splat_i(sba) + _splat_i(i) * 3
                ibb = _splat_i(sbb) + _splat_i(i) * 3
                pa = plsc.load_gather(subtree_v, [iba])
                la = plsc.load_gather(subtree_v, [iba + 1])
                ra = plsc.load_gather(subtree_v, [iba + 2])
                pb = plsc.load_gather(subtree_v, [ibb])
                lb = plsc.load_gather(subtree_v, [ibb + 1])
                rb = plsc.load_gather(subtree_v, [ibb + 2])

                rla = _splat_i(row0a) + la
                rra = _splat_i(row0a) + ra
                rlb = _splat_i(row0b) + lb
                rrb = _splat_i(row0b) + rb

                acca = [btree_v[pl.ds(c * _L, _L)] for c in range(4)]
                accb = [btree_v[pl.ds(c * _L, _L)] for c in range(4)]

                kblk = 4

                def block_loads(kb):
                    wv = [wlr_v[kb + j, pl.ds(c * _L, _L)]
                          for j in range(kblk) for c in range(4)]
                    sa, sb = [], []
                    for j in range(kblk):
                        k = kb + j
                        if k < _E:
                            va, vb, col = rla, rlb, k
                        else:
                            va, vb, col = rra, rrb, k - _E
                        sa.append(plsc.load_gather(reps_v, [va, _splat_i(col)]))
                        sb.append(plsc.load_gather(reps_v, [vb, _splat_i(col)]))
                    return wv, sa, sb

                cur = block_loads(0)
                for kb in range(0, 2 * _E, kblk):
                    nxt = block_loads(kb + kblk) if kb + kblk < 2 * _E else None
                    wv, sa, sb = cur
                    for j in range(kblk):
                        for c in range(4):
                            acca[c] = acca[c] + wv[j * 4 + c] * sa[j]
                            accb[c] = accb[c] + wv[j * 4 + c] * sb[j]
                    cur = nxt

                iv = _splat_i(i)
                conda = jnp.logical_and(la != ra, iv < s0v)
                condb = jnp.logical_and(lb != rb, iv < s1v)
                rpa = _splat_i(row0a) + pa
                rpb = _splat_i(row0b) + pb
                for c in range(4):
                    plsc.store_scatter(reps_v, [rpa, iota + c * _L],
                                       jnp.maximum(acca[c], 0.0), mask=conda)
                    plsc.store_scatter(reps_v, [rpb, iota + c * _L],
                                       jnp.maximum(accb[c], 0.0), mask=condb)
                return carry3

            lax.fori_loop(0, jnp.maximum(s0, s1), step_body, 0)

            for b, row0 in ((2 * p, row0a), (2 * p + 1, row0b)):
                b0v = _splat_i(row0)
                acc_o = bcls_v[...]
                for k in range(_E):
                    xk = plsc.load_gather(reps_v, [b0v, _splat_i(k)])
                    acc_o = acc_o + wcls_v[k, pl.ds(0, _L)] * xk
                plsc.store_scatter(out_stage, [_splat_i(g * _G + b), iota],
                                   acc_o)
            return carry2

        lax.fori_loop(0, _G // 2, pair_body, 0)
        return carry

    lax.fori_loop(0, _NG, group_body, 0)
    pltpu.sync_copy(out_stage, out_hbm.at[pl.ds(wid * _EPT, _EPT)])


@jax.jit
def _run(subtree_flat, embind_g, slens_pad, emb_table, w_lr, b_tree,
         wcls_pad, bcls_pad):
    mesh = plsc.VectorSubcoreMesh(core_axis_name="c", subcore_axis_name="s")
    f = functools.partial(
        pl.kernel,
        mesh=mesh,
        compiler_params=pltpu.CompilerParams(needs_layout_passes=False,
                                             use_tc_tiling_on_sc=False),
        out_type=jax.ShapeDtypeStruct((_B, _L), jnp.float32),
        scratch_types=[
            pltpu.VMEM((_G * _N, _E), jnp.float32),
            pltpu.VMEM((_G * _N * 3 + _L,), jnp.int32),
            pltpu.VMEM((16, 100), jnp.int32),
            pltpu.VMEM((2 * _L,), jnp.int32),
            pltpu.VMEM((2 * _E, _E), jnp.float32),
            pltpu.VMEM((_E,), jnp.float32),
            pltpu.VMEM((_E, _L), jnp.float32),
            pltpu.VMEM((_L,), jnp.float32),
            pltpu.VMEM((_EPT, _L), jnp.float32),
            pltpu.SemaphoreType.DMA,
        ],
    )(_tree_kernel)
    return f(subtree_flat, embind_g, slens_pad, emb_table, w_lr, b_tree,
             wcls_pad, bcls_pad)


def kernel(subtree_batch, subtree_lens_batch, emb_ind_batch, emb_table,
           W_tree, b_tree, W_cls, b_cls):
    slens = subtree_lens_batch.astype(jnp.int32)
    order = jnp.argsort(slens)
    q = jnp.arange(_B // 2, dtype=jnp.int32)
    npos_even = (q % _NTILES) * _EPT + (q // _NTILES) * 2
    perm = jnp.zeros((_B,), jnp.int32)
    perm = perm.at[npos_even].set(order[0::2])
    perm = perm.at[npos_even + 1].set(order[1::2])
    inv = jnp.zeros((_B,), jnp.int32).at[perm].set(
        jnp.arange(_B, dtype=jnp.int32))

    subtree_flat = subtree_batch.astype(jnp.int32)[perm].reshape(-1)
    embind_g = emb_ind_batch.astype(jnp.int32)[perm].reshape(
        _B // _G, 16, 100)
    slens_p = slens[perm].reshape(_B // _G, _G)
    slens_pad = jnp.pad(slens_p, ((0, 0), (0, _L - _G)))
    w_lr = W_tree.T
    wcls_pad = jnp.pad(W_cls, ((0, _L - 5), (0, 0))).T
    bcls_pad = jnp.pad(b_cls, (0, _L - 5))
    out = _run(subtree_flat, embind_g, slens_pad, emb_table, w_lr,
               b_tree, wcls_pad, bcls_pad)
    return out[inv, :5]


# speedup vs baseline: 1.0083x; 1.0083x over previous
import functools

import jax
import jax.numpy as jnp
from jax import lax
from jax.experimental import pallas as pl
from jax.experimental.pallas import tpu as pltpu
from jax.experimental.pallas import tpu_sc as plsc

_B = 1024
_N = 200
_E = 64
_L = 16
_NTILES = 32
_EPT = _B // _NTILES
_G = 8
_NG = _EPT // _G


def _splat_i(x):
    return jnp.full((_L,), x, dtype=jnp.int32)


def _tree_kernel(subtree_hbm, embind_hbm, slens_hbm, emb_hbm, wlr_hbm,
                 btree_hbm, wcls_hbm, bcls_hbm, out_hbm,
                 reps_v, subtree_v, embind_v, slens_v, wlr_v, btree_v,
                 wcls_v, bcls_v, out_stage, sem):
    nc = 2
    wid = lax.axis_index("s") * nc + lax.axis_index("c")

    pltpu.sync_copy(wlr_hbm, wlr_v)
    pltpu.sync_copy(btree_hbm, btree_v)
    pltpu.sync_copy(wcls_hbm, wcls_v)
    pltpu.sync_copy(bcls_hbm, bcls_v)

    iota = lax.iota(jnp.int32, _L)

    def group_body(g, carry):
        e0 = wid * _EPT + g * _G
        grp = wid * _NG + g

        pltpu.sync_copy(subtree_hbm.at[pl.ds(e0 * (_N * 3), _G * _N * 3)],
                        subtree_v.at[pl.ds(0, _G * _N * 3)])
        pltpu.sync_copy(embind_hbm.at[grp], embind_v)
        pltpu.sync_copy(slens_hbm.at[grp], slens_v.at[pl.ds(0, _L)])

        copies = [pltpu.async_copy(emb_hbm.at[embind_v.at[j]],
                                   reps_v.at[pl.ds(j * 100, 100)], sem)
                  for j in range(16)]
        for cp in copies:
            cp.wait()

        def pair_body(p, carry2):
            sl = slens_v[pl.ds(2 * p, _L)]
            s0 = sl[0]
            s1 = sl[1]
            row0a = (2 * p) * _N
            row0b = (2 * p + 1) * _N
            sba = (2 * p) * (_N * 3)
            sbb = (2 * p + 1) * (_N * 3)
            s0v = _splat_i(s0)
            s1v = _splat_i(s1)

            def step_body(i, carry3):
                iba = _splat_i(sba) + _splat_i(i) * 3
                ibb = _splat_i(sbb) + _splat_i(i) * 3
                pa = plsc.load_gather(subtree_v, [iba])
                la = plsc.load_gather(subtree_v, [iba + 1])
                ra = plsc.load_gather(subtree_v, [iba + 2])
                pb = plsc.load_gather(subtree_v, [ibb])
                lb = plsc.load_gather(subtree_v, [ibb + 1])
                rb = plsc.load_gather(subtree_v, [ibb + 2])

                rla = _splat_i(row0a) + la
                rra = _splat_i(row0a) + ra
                rlb = _splat_i(row0b) + lb
                rrb = _splat_i(row0b) + rb

                acca = [btree_v[pl.ds(c * _L, _L)] for c in range(4)]
                accb = [btree_v[pl.ds(c * _L, _L)] for c in range(4)]

                kblk = 4

                def block_loads(kb):
                    wv = [wlr_v[kb + j, pl.ds(c * _L, _L)]
                          for j in range(kblk) for c in range(4)]
                    sa, sb = [], []
                    for j in range(kblk):
                        k = kb + j
                        if k < _E:
                            va, vb, col = rla, rlb, k
                        else:
                            va, vb, col = rra, rrb, k - _E
                        sa.append(plsc.load_gather(reps_v, [va, _splat_i(col)]))
                        sb.append(plsc.load_gather(reps_v, [vb, _splat_i(col)]))
                    return wv, sa, sb

                cur = block_loads(0)
                for kb in range(0, 2 * _E, kblk):
                    nxt = block_loads(kb + kblk) if kb + kblk < 2 * _E else None
                    wv, sa, sb = cur
                    for j in range(kblk):
                        for c in range(4):
                            acca[c] = acca[c] + wv[j * 4 + c] * sa[j]
                            accb[c] = accb[c] + wv[j * 4 + c] * sb[j]
                    cur = nxt

                iv = _splat_i(i)
                conda = jnp.logical_and(la != ra, iv < s0v)
                condb = jnp.logical_and(lb != rb, iv < s1v)
                rpa = _splat_i(row0a) + pa
                rpb = _splat_i(row0b) + pb
                for c in range(4):
                    plsc.store_scatter(reps_v, [rpa, iota + c * _L],
                                       jnp.maximum(acca[c], 0.0), mask=conda)
                    plsc.store_scatter(reps_v, [rpb, iota + c * _L],
                                       jnp.maximum(accb[c], 0.0), mask=condb)
                return carry3

            lax.fori_loop(0, jnp.maximum(s0, s1), step_body, 0)

            for b, row0 in ((2 * p, row0a), (2 * p + 1, row0b)):
                b0v = _splat_i(row0)
                acc_o = bcls_v[...]
                for k in range(_E):
                    xk = plsc.load_gather(reps_v, [b0v, _splat_i(k)])
                    acc_o = acc_o + wcls_v[k, pl.ds(0, _L)] * xk
                plsc.store_scatter(out_stage, [_splat_i(g * _G + b), iota],
                                   acc_o)
            return carry2

        lax.fori_loop(0, _G // 2, pair_body, 0)
        return carry

    lax.fori_loop(0, _NG, group_body, 0)
    pltpu.sync_copy(out_stage, out_hbm.at[pl.ds(wid * _EPT, _EPT)])


@jax.jit
def _run(subtree_flat, embind_g, slens_pad, emb_table, w_lr, b_tree,
         wcls_pad, bcls_pad):
    mesh = plsc.VectorSubcoreMesh(core_axis_name="c", subcore_axis_name="s")
    f = functools.partial(
        pl.kernel,
        mesh=mesh,
        compiler_params=pltpu.CompilerParams(needs_layout_passes=False,
                                             use_tc_tiling_on_sc=False),
        out_type=jax.ShapeDtypeStruct((_B, _L), jnp.float32),
        scratch_types=[
            pltpu.VMEM((_G * _N, _E), jnp.float32),
            pltpu.VMEM((_G * _N * 3 + _L,), jnp.int32),
            pltpu.VMEM((16, 100), jnp.int32),
            pltpu.VMEM((2 * _L,), jnp.int32),
            pltpu.VMEM((2 * _E, _E), jnp.float32),
            pltpu.VMEM((_E,), jnp.float32),
            pltpu.VMEM((_E, _L), jnp.float32),
            pltpu.VMEM((_L,), jnp.float32),
            pltpu.VMEM((_EPT, _L), jnp.float32),
            pltpu.SemaphoreType.DMA,
        ],
    )(_tree_kernel)
    return f(subtree_flat, embind_g, slens_pad, emb_table, w_lr, b_tree,
             wcls_pad, bcls_pad)


def kernel(subtree_batch, subtree_lens_batch, emb_ind_batch, emb_table,
           W_tree, b_tree, W_cls, b_cls):
    slens = subtree_lens_batch.astype(jnp.int32)
    order = jnp.argsort(slens)
    q = jnp.arange(_B // 2, dtype=jnp.int32)
    npos_even = (q % _NTILES) * _EPT + (q // _NTILES) * 2
    perm = jnp.zeros((_B,), jnp.int32)
    perm = perm.at[npos_even].set(order[0::2])
    perm = perm.at[npos_even + 1].set(order[1::2])
    inv = jnp.zeros((_B,), jnp.int32).at[perm].set(
        jnp.arange(_B, dtype=jnp.int32))

    subtree_flat = subtree_batch.astype(jnp.int32)[perm].reshape(-1)
    embind_g = emb_ind_batch.astype(jnp.int32)[perm].reshape(
        _B // _G, 16, 100)
    slens_p = slens[perm].reshape(_B // _G, _G)
    slens_pad = jnp.pad(slens_p, ((0, 0), (0, _L - _G)))
    w_lr = W_tree.T
    wcls_pad = jnp.pad(W_cls, ((0, _L - 5), (0, 0))).T
    bcls_pad = jnp.pad(b_cls, (0, _L - 5))
    out = _run(subtree_flat, embind_g, slens_pad, emb_table, w_lr,
               b_tree, wcls_pad, bcls_pad)
    return out[inv, :5]


# speedup vs baseline: 4.6928x; 4.6542x over previous
import functools

import jax
import jax.numpy as jnp
from jax import lax
from jax.experimental import pallas as pl
from jax.experimental.pallas import tpu as pltpu
from jax.experimental.pallas import tpu_sc as plsc

_B = 1024
_N = 200
_E = 64
_L = 16
_NTILES = 32
_EPT = _B // _NTILES
_G = 8
_NG = _EPT // _G


def _splat_i(x):
    return jnp.full((_L,), x, dtype=jnp.int32)


def _tree_kernel(subtree_hbm, embind_hbm, slens_hbm, emb_hbm, wlr_hbm,
                 btree_hbm, wcls_hbm, bcls_hbm, out_hbm,
                 reps_v, subtree_v, embind_v, slens_v, wlr_v, btree_v,
                 wcls_v, bcls_v, out_stage, sem):
    nc = 2
    wid = lax.axis_index("s") * nc + lax.axis_index("c")

    pltpu.sync_copy(wlr_hbm, wlr_v)
    pltpu.sync_copy(btree_hbm, btree_v)
    pltpu.sync_copy(wcls_hbm, wcls_v)
    pltpu.sync_copy(bcls_hbm, bcls_v)

    iota = lax.iota(jnp.int32, _L)

    def group_body(g, carry):
        e0 = wid * _EPT + g * _G
        grp = wid * _NG + g

        pltpu.sync_copy(subtree_hbm.at[pl.ds(e0 * (_N * 3), _G * _N * 3)],
                        subtree_v.at[pl.ds(0, _G * _N * 3)])
        pltpu.sync_copy(embind_hbm.at[grp], embind_v)
        pltpu.sync_copy(slens_hbm.at[grp], slens_v.at[pl.ds(0, _L)])

        copies = [pltpu.async_copy(emb_hbm.at[embind_v.at[j]],
                                   reps_v.at[pl.ds(j * 100, 100)], sem)
                  for j in range(16)]
        for cp in copies:
            cp.wait()

        def pair_body(p, carry2):
            sl = slens_v[pl.ds(2 * p, _L)]
            s0 = sl[0]
            s1 = sl[1]
            row0a = (2 * p) * _N
            row0b = (2 * p + 1) * _N
            sba = (2 * p) * (_N * 3)
            sbb = (2 * p + 1) * (_N * 3)
            s0v = _splat_i(s0)
            s1v = _splat_i(s1)

            def step_body(i, carry3):
                iba = _splat_i(sba) + _splat_i(i) * 3
                ibb = _splat_i(sbb) + _splat_i(i) * 3
                pa = plsc.load_gather(subtree_v, [iba])
                la = plsc.load_gather(subtree_v, [iba + 1])
                ra = plsc.load_gather(subtree_v, [iba + 2])
                pb = plsc.load_gather(subtree_v, [ibb])
                lb = plsc.load_gather(subtree_v, [ibb + 1])
                rb = plsc.load_gather(subtree_v, [ibb + 2])

                rla = _splat_i(row0a) + la
                rra = _splat_i(row0a) + ra
                rlb = _splat_i(row0b) + lb
                rrb = _splat_i(row0b) + rb

                acca = [btree_v[pl.ds(c * _L, _L)] for c in range(4)]
                accb = [btree_v[pl.ds(c * _L, _L)] for c in range(4)]

                kblk = 4

                def block_loads(kb):
                    wv = [wlr_v[kb + j, pl.ds(c * _L, _L)]
                          for j in range(kblk) for c in range(4)]
                    sa, sb = [], []
                    for j in range(kblk):
                        k = kb + j
                        if k < _E:
                            va, vb, col = rla, rlb, k
                        else:
                            va, vb, col = rra, rrb, k - _E
                        sa.append(plsc.load_gather(reps_v, [va, _splat_i(col)]))
                        sb.append(plsc.load_gather(reps_v, [vb, _splat_i(col)]))
                    return wv, sa, sb

                cur = block_loads(0)
                for kb in range(0, 2 * _E, kblk):
                    nxt = block_loads(kb + kblk) if kb + kblk < 2 * _E else None
                    wv, sa, sb = cur
                    for j in range(kblk):
                        for c in range(4):
                            acca[c] = acca[c] + wv[j * 4 + c] * sa[j]
                            accb[c] = accb[c] + wv[j * 4 + c] * sb[j]
                    cur = nxt

                iv = _splat_i(i)
                conda = jnp.logical_and(la != ra, iv < s0v)
                condb = jnp.logical_and(lb != rb, iv < s1v)
                rpa = _splat_i(row0a) + pa
                rpb = _splat_i(row0b) + pb
                for c in range(4):
                    plsc.store_scatter(reps_v, [rpa, iota + c * _L],
                                       jnp.maximum(acca[c], 0.0), mask=conda)
                    plsc.store_scatter(reps_v, [rpb, iota + c * _L],
                                       jnp.maximum(accb[c], 0.0), mask=condb)
                return carry3

            lax.fori_loop(0, jnp.maximum(s0, s1), step_body, 0)

            for b, row0 in ((2 * p, row0a), (2 * p + 1, row0b)):
                b0v = _splat_i(row0)
                acc_o = bcls_v[...]
                for k in range(_E):
                    xk = plsc.load_gather(reps_v, [b0v, _splat_i(k)])
                    acc_o = acc_o + wcls_v[k, pl.ds(0, _L)] * xk
                plsc.store_scatter(out_stage, [_splat_i(g * _G + b), iota],
                                   acc_o)
            return carry2

        lax.fori_loop(0, _G // 2, pair_body, 0)
        return carry

    lax.fori_loop(0, _NG, group_body, 0)
    pltpu.sync_copy(out_stage, out_hbm.at[pl.ds(wid * _EPT, _EPT)])


@jax.jit
def _run(subtree_flat, embind_g, slens_pad, emb_table, w_lr, b_tree,
         wcls_pad, bcls_pad):
    mesh = plsc.VectorSubcoreMesh(core_axis_name="c", subcore_axis_name="s")
    f = functools.partial(
        pl.kernel,
        mesh=mesh,
        compiler_params=pltpu.CompilerParams(needs_layout_passes=False,
                                             use_tc_tiling_on_sc=False),
        out_type=jax.ShapeDtypeStruct((_B, _L), jnp.float32),
        scratch_types=[
            pltpu.VMEM((_G * _N, _E), jnp.float32),
            pltpu.VMEM((_G * _N * 3 + _L,), jnp.int32),
            pltpu.VMEM((16, 100), jnp.int32),
            pltpu.VMEM((2 * _L,), jnp.int32),
            pltpu.VMEM((2 * _E, _E), jnp.float32),
            pltpu.VMEM((_E,), jnp.float32),
            pltpu.VMEM((_E, _L), jnp.float32),
            pltpu.VMEM((_L,), jnp.float32),
            pltpu.VMEM((_EPT, _L), jnp.float32),
            pltpu.SemaphoreType.DMA,
        ],
    )(_tree_kernel)
    return f(subtree_flat, embind_g, slens_pad, emb_table, w_lr, b_tree,
             wcls_pad, bcls_pad)


def kernel(subtree_batch, subtree_lens_batch, emb_ind_batch, emb_table,
           W_tree, b_tree, W_cls, b_cls):
    slens = subtree_lens_batch.astype(jnp.int32)
    order = jnp.argsort(slens)
    perm = order.reshape(16, _NTILES, 2).transpose(1, 0, 2).reshape(-1)
    pmat = (perm[:, None] == jnp.arange(_B)[None, :]).astype(jnp.float32)

    packed = jnp.concatenate(
        [subtree_batch.astype(jnp.int32).reshape(_B, _N * 3),
         emb_ind_batch.astype(jnp.int32),
         slens[:, None]], axis=1)
    packed_p = jnp.dot(pmat, packed.astype(jnp.float32),
                       precision=lax.Precision.HIGHEST,
                       preferred_element_type=jnp.float32).astype(jnp.int32)

    subtree_flat = packed_p[:, :_N * 3].reshape(-1)
    embind_g = packed_p[:, _N * 3:_N * 4].reshape(_B // _G, 16, 100)
    slens_p = packed_p[:, _N * 4].reshape(_B // _G, _G)
    slens_pad = jnp.pad(slens_p, ((0, 0), (0, _L - _G)))
    w_lr = W_tree.T
    wcls_pad = jnp.pad(W_cls, ((0, _L - 5), (0, 0))).T
    bcls_pad = jnp.pad(b_cls, (0, _L - 5))
    out = _run(subtree_flat, embind_g, slens_pad, emb_table, w_lr,
               b_tree, wcls_pad, bcls_pad)
    return jnp.dot(pmat.T, out[:, :5], precision=lax.Precision.HIGHEST,
                   preferred_element_type=jnp.float32)
